# Initial kernel scaffold; baseline (speedup 1.0000x reference)
import jax
import jax.numpy as jnp
from jax.experimental import pallas as pl


def kernel(x, edge_index, edge_attr, batch, W1, b1, W2, b2, W3, b3, W_root, b_conv, W_gat, att_src, att_dst, b_gat, W_fc1, b_fc1):
    raise NotImplementedError("write your pallas kernel here")


# speedup vs baseline: 6.1670x; 6.1670x over previous
import functools

import jax
import jax.numpy as jnp
from jax import lax
from jax.experimental import pallas as pl
from jax.experimental.pallas import tpu as pltpu
from jax.experimental.pallas import tpu_sc as plsc

N = 10000
E = 160000
F_IN = 16
F_EDGE = 16
HID = 16
OUT = 64

NC = 2
NS = 16
NW = NC * NS
CHUNK = 128
NCHUNKS = E // CHUNK
CPW = -(-NCHUNKS // NW)
RPS = N // NS

def _worker_id():
    return lax.axis_index("s") * NC + lax.axis_index("c")


@functools.cache
def _sc_kernels():
    mesh = plsc.VectorSubcoreMesh(
        core_axis_name="c", subcore_axis_name="s", num_cores=NC, num_subcores=NS)

    @functools.partial(
        pl.kernel,
        out_type=jax.ShapeDtypeStruct((E, F_IN), jnp.float32),
        mesh=mesh,
        compiler_params=pltpu.CompilerParams(use_tc_tiling_on_sc=False, needs_layout_passes=False),
        scratch_types=[
            pltpu.VMEM((CHUNK,), jnp.int32),
            pltpu.VMEM((CHUNK, F_IN), jnp.float32),
            pltpu.SemaphoreType.DMA,
        ],
    )
    def _gather_rows(x_hbm, idx_hbm, out_hbm, idx_v, rows_v, sem):
        w = _worker_id()

        def body(j, carry):
            cid = w + NW * j

            @pl.when(cid < NCHUNKS)
            def _():
                base = cid * CHUNK
                pltpu.sync_copy(idx_hbm.at[pl.ds(base, CHUNK)], idx_v)
                pltpu.async_copy(x_hbm.at[idx_v], rows_v, sem).wait()
                pltpu.sync_copy(rows_v, out_hbm.at[pl.ds(base, CHUNK)])

            return carry

        lax.fori_loop(0, CPW, body, 0)

    @functools.partial(
        pl.kernel,
        out_type=(
            jax.ShapeDtypeStruct((E, HID), jnp.float32),
            jax.ShapeDtypeStruct((E,), jnp.float32),
        ),
        mesh=mesh,
        compiler_params=pltpu.CompilerParams(use_tc_tiling_on_sc=False, needs_layout_passes=False),
        scratch_types=[
            pltpu.VMEM((N,), jnp.float32),
            pltpu.VMEM((N,), jnp.float32),
            pltpu.VMEM((CHUNK,), jnp.int32),
            pltpu.VMEM((CHUNK,), jnp.int32),
            pltpu.VMEM((CHUNK, HID), jnp.float32),
            pltpu.VMEM((CHUNK,), jnp.float32),
            pltpu.SemaphoreType.DMA,
        ],
    )
    def _gather_gat(xw_hbm, as_hbm, ad_hbm, src_hbm, dst_hbm, xsrc_out, esum_out,
                    a_s_v, a_d_v, idxs_v, idxd_v, rows_v, es_v, sem):
        w = _worker_id()
        pltpu.sync_copy(as_hbm, a_s_v)
        pltpu.sync_copy(ad_hbm, a_d_v)

        def body(j, carry):
            cid = w + NW * j

            @pl.when(cid < NCHUNKS)
            def _():
                base = cid * CHUNK
                pltpu.sync_copy(src_hbm.at[pl.ds(base, CHUNK)], idxs_v)
                pltpu.sync_copy(dst_hbm.at[pl.ds(base, CHUNK)], idxd_v)
                pltpu.async_copy(xw_hbm.at[idxs_v], rows_v, sem).wait()
                pltpu.sync_copy(rows_v, xsrc_out.at[pl.ds(base, CHUNK)])
                for k in range(CHUNK // 16):
                    si = idxs_v[pl.ds(k * 16, 16)]
                    di = idxd_v[pl.ds(k * 16, 16)]
                    vs = plsc.load_gather(a_s_v, [si])
                    vd = plsc.load_gather(a_d_v, [di])
                    es_v[pl.ds(k * 16, 16)] = vs + vd
                pltpu.sync_copy(es_v, esum_out.at[pl.ds(base, CHUNK)])

            return carry

        lax.fori_loop(0, CPW, body, 0)

    @functools.partial(
        pl.kernel,
        out_type=jax.ShapeDtypeStruct((NC, N, F_IN), jnp.float32),
        mesh=mesh,
        compiler_params=pltpu.CompilerParams(use_tc_tiling_on_sc=False, needs_layout_passes=False),
        scratch_types=[
            pltpu.VMEM((CHUNK,), jnp.int32),
            pltpu.VMEM((CHUNK, F_IN), jnp.float32),
            pltpu.VMEM_SHARED((N, F_IN), jnp.float32),
            pltpu.SemaphoreType.DMA,
        ],
    )
    def _scatter_add1(zeros_hbm, dst_hbm, rows_hbm, out_hbm, idx_v, rows_v, acc, sem):
        c = lax.axis_index("c")
        s = lax.axis_index("s")
        w = s * NC + c
        pltpu.sync_copy(zeros_hbm.at[pl.ds(s * RPS, RPS)], acc.at[pl.ds(s * RPS, RPS)])
        plsc.subcore_barrier()

        def body(j, carry):
            cid = w + NW * j

            @pl.when(cid < NCHUNKS)
            def _():
                base = cid * CHUNK
                pltpu.sync_copy(dst_hbm.at[pl.ds(base, CHUNK)], idx_v)
                pltpu.sync_copy(rows_hbm.at[pl.ds(base, CHUNK)], rows_v)
                pltpu.sync_copy(rows_v, acc.at[idx_v], add=True)

            return carry

        lax.fori_loop(0, CPW, body, 0)
        plsc.subcore_barrier()
        pltpu.sync_copy(acc.at[pl.ds(s * RPS, RPS)],
                        out_hbm.at[c, pl.ds(s * RPS, RPS)])

    @functools.partial(
        pl.kernel,
        out_type=(
            jax.ShapeDtypeStruct((NC, N, HID), jnp.float32),
            jax.ShapeDtypeStruct((NC, N, HID), jnp.float32),
        ),
        mesh=mesh,
        compiler_params=pltpu.CompilerParams(use_tc_tiling_on_sc=False, needs_layout_passes=False),
        scratch_types=[
            pltpu.VMEM((CHUNK,), jnp.int32),
            pltpu.VMEM((CHUNK, HID), jnp.float32),
            pltpu.VMEM((CHUNK, HID), jnp.float32),
            pltpu.VMEM_SHARED((N, HID), jnp.float32),
            pltpu.VMEM_SHARED((N, HID), jnp.float32),
            pltpu.SemaphoreType.DMA,
        ],
    )
    def _scatter_add2(zeros_hbm, dst_hbm, rowsa_hbm, rowsb_hbm, outa_hbm, outb_hbm,
                      idx_v, rowsa_v, rowsb_v, acca, accb, sem):
        c = lax.axis_index("c")
        s = lax.axis_index("s")
        w = s * NC + c
        pltpu.sync_copy(zeros_hbm.at[pl.ds(s * RPS, RPS)], acca.at[pl.ds(s * RPS, RPS)])
        pltpu.sync_copy(zeros_hbm.at[pl.ds(s * RPS, RPS)], accb.at[pl.ds(s * RPS, RPS)])
        plsc.subcore_barrier()

        def body(j, carry):
            cid = w + NW * j

            @pl.when(cid < NCHUNKS)
            def _():
                base = cid * CHUNK
                pltpu.sync_copy(dst_hbm.at[pl.ds(base, CHUNK)], idx_v)
                pltpu.sync_copy(rowsa_hbm.at[pl.ds(base, CHUNK)], rowsa_v)
                pltpu.sync_copy(rowsb_hbm.at[pl.ds(base, CHUNK)], rowsb_v)
                pltpu.sync_copy(rowsa_v, acca.at[idx_v], add=True)
                pltpu.sync_copy(rowsb_v, accb.at[idx_v], add=True)

            return carry

        lax.fori_loop(0, CPW, body, 0)
        plsc.subcore_barrier()
        pltpu.sync_copy(acca.at[pl.ds(s * RPS, RPS)],
                        outa_hbm.at[c, pl.ds(s * RPS, RPS)])
        pltpu.sync_copy(accb.at[pl.ds(s * RPS, RPS)],
                        outb_hbm.at[c, pl.ds(s * RPS, RPS)])

    return _gather_rows, _gather_gat, _scatter_add1, _scatter_add2


BE = 2000


def _mlp_body(ea_ref, xs_ref, w1_ref, b1_ref, w2_ref, b2_ref, w3_ref, b3_ref,
              q_ref, p_ref, out_ref):
    f32 = jnp.float32
    h1 = jnp.maximum(
        jnp.dot(ea_ref[...], w1_ref[...], preferred_element_type=f32) + b1_ref[...], 0.0)
    h2 = jnp.maximum(
        jnp.dot(h1, w2_ref[...], preferred_element_type=f32) + b2_ref[...], 0.0)
    w = jnp.dot(h2, w3_ref[...], preferred_element_type=f32) + b3_ref[...]
    xr = jnp.dot(xs_ref[...], q_ref[...], preferred_element_type=f32)
    out_ref[...] = jnp.dot(w * xr, p_ref[...], preferred_element_type=f32)


_edge_mlp = pl.pallas_call(
    _mlp_body,
    grid=(E // BE,),
    in_specs=[
        pl.BlockSpec((BE, F_EDGE), lambda i: (i, 0)),
        pl.BlockSpec((BE, F_IN), lambda i: (i, 0)),
        pl.BlockSpec((F_EDGE, 128), lambda i: (0, 0)),
        pl.BlockSpec((1, 128), lambda i: (0, 0)),
        pl.BlockSpec((128, 64), lambda i: (0, 0)),
        pl.BlockSpec((1, 64), lambda i: (0, 0)),
        pl.BlockSpec((64, F_IN * HID), lambda i: (0, 0)),
        pl.BlockSpec((1, F_IN * HID), lambda i: (0, 0)),
        pl.BlockSpec((F_IN, F_IN * HID), lambda i: (0, 0)),
        pl.BlockSpec((F_IN * HID, HID), lambda i: (0, 0)),
    ],
    out_specs=pl.BlockSpec((BE, HID), lambda i: (i, 0)),
    out_shape=jax.ShapeDtypeStruct((E, HID), jnp.float32),
)


def _node_body(x_ref, a0_ref, a1_ref, wr_ref, bc_ref, wg_ref, asv_ref, adv_ref,
               xw_ref, as_ref, ad_ref, c_ref, num0_ref, den0_ref):
    f32 = jnp.float32
    agg = a0_ref[...] + a1_ref[...]
    x1 = jnp.maximum(
        jnp.dot(x_ref[...], wr_ref[...], preferred_element_type=f32) + agg + bc_ref[...],
        0.0)
    xw = jnp.dot(x1, wg_ref[...], preferred_element_type=f32)
    a_s = jnp.sum(xw * asv_ref[...], axis=1, keepdims=True)
    a_d = jnp.sum(xw * adv_ref[...], axis=1, keepdims=True)
    cval = jnp.max(a_s) + jnp.max(a_d)
    z = a_s + a_d
    ps = jnp.exp(jnp.where(z >= 0, z, 0.2 * z) - cval)
    xw_ref[...] = xw
    as_ref[...] = a_s
    ad_ref[...] = a_d
    c_ref[...] = jnp.reshape(cval, (1, 1))
    num0_ref[...] = ps * xw
    den0_ref[...] = ps


_node_stage = pl.pallas_call(
    _node_body,
    out_shape=(
        jax.ShapeDtypeStruct((N, HID), jnp.float32),
        jax.ShapeDtypeStruct((N, 1), jnp.float32),
        jax.ShapeDtypeStruct((N, 1), jnp.float32),
        jax.ShapeDtypeStruct((1, 1), jnp.float32),
        jax.ShapeDtypeStruct((N, HID), jnp.float32),
        jax.ShapeDtypeStruct((N, 1), jnp.float32),
    ),
)


BG = 4000


def _gat_edge_body(es_ref, c_ref, xs_ref, m2_ref, pr_ref):
    z = es_ref[...]
    e = jnp.where(z >= 0, z, 0.2 * z)
    p = jnp.exp(e - c_ref[0, 0])
    m2_ref[...] = p * xs_ref[...]
    pr_ref[...] = p * jnp.ones_like(xs_ref[...])


_gat_edge = pl.pallas_call(
    _gat_edge_body,
    grid=(E // BG,),
    in_specs=[
        pl.BlockSpec((BG, 1), lambda i: (i, 0)),
        pl.BlockSpec((1, 1), lambda i: (0, 0)),
        pl.BlockSpec((BG, HID), lambda i: (i, 0)),
    ],
    out_specs=[
        pl.BlockSpec((BG, HID), lambda i: (i, 0)),
        pl.BlockSpec((BG, HID), lambda i: (i, 0)),
    ],
    out_shape=(
        jax.ShapeDtypeStruct((E, HID), jnp.float32),
        jax.ShapeDtypeStruct((E, HID), jnp.float32),
    ),
)


def _final_body(n0_ref, n1_ref, n2_ref, d0_ref, d1_ref, d2_ref, bg_ref, wf_ref,
                bf_ref, out_ref):
    f32 = jnp.float32
    num = n0_ref[...] + n1_ref[...] + n2_ref[...]
    den = d0_ref[...] + d1_ref[...] + d2_ref[...]
    out_g = num / den + bg_ref[...]
    x2 = jnp.maximum(out_g, 0.0)
    out_ref[...] = jnp.maximum(
        jnp.dot(x2, wf_ref[...], preferred_element_type=f32) + bf_ref[...], 0.0)


_final_stage = pl.pallas_call(
    _final_body,
    out_shape=jax.ShapeDtypeStruct((N, OUT), jnp.float32),
)


def kernel(x, edge_index, edge_attr, batch, W1, b1, W2, b2, W3, b3,
           W_root, b_conv, W_gat, att_src, att_dst, b_gat, W_fc1, b_fc1):
    _gather_rows, _gather_gat, _scatter_add1, _scatter_add2 = _sc_kernels()
    src = edge_index[0]
    dst = edge_index[1]
    zeros_n = jnp.zeros((N, F_IN), jnp.float32)
    jcol = jnp.arange(F_IN * HID)
    qmat = (jnp.arange(F_IN)[:, None] == (jcol[None, :] // HID)).astype(jnp.float32)
    pmat = ((jcol[:, None] % HID) == jnp.arange(HID)[None, :]).astype(jnp.float32)

    xs = _gather_rows(x, src)
    msg = _edge_mlp(edge_attr, xs, W1, b1.reshape(1, -1), W2, b2.reshape(1, -1),
                    W3, b3.reshape(1, -1), qmat, pmat)
    aggp = _scatter_add1(zeros_n, dst, msg)
    xw, a_s, a_d, cvl, num0, den0 = _node_stage(
        x, aggp[0], aggp[1], W_root, b_conv.reshape(1, -1), W_gat,
        att_src.reshape(1, -1), att_dst.reshape(1, -1))
    xw_src, esum = _gather_gat(xw, a_s.reshape(N), a_d.reshape(N), src, dst)
    m2, p_rep = _gat_edge(esum.reshape(E, 1), cvl, xw_src)
    nump, denp = _scatter_add2(zeros_n, dst, m2, p_rep)
    out = _final_stage(num0, nump[0], nump[1], den0, denp[0], denp[1],
                       b_gat.reshape(1, -1), W_fc1, b_fc1.reshape(1, -1))
    return out


# speedup vs baseline: 10.2449x; 1.6612x over previous
import functools

import jax
import jax.numpy as jnp
from jax import lax
from jax.experimental import pallas as pl
from jax.experimental.pallas import tpu as pltpu
from jax.experimental.pallas import tpu_sc as plsc

N = 10000
E = 160000
F_IN = 16
F_EDGE = 16
HID = 16
OUT = 64

NC = 2
NS = 16
NW = NC * NS
CHUNK = 128
NCHUNKS = E // CHUNK
CPW = -(-NCHUNKS // NW)
RPS = N // NS

def _worker_id():
    return lax.axis_index("s") * NC + lax.axis_index("c")


@functools.cache
def _sc_kernels():
    mesh = plsc.VectorSubcoreMesh(
        core_axis_name="c", subcore_axis_name="s", num_cores=NC, num_subcores=NS)

    @functools.partial(
        pl.kernel,
        out_type=jax.ShapeDtypeStruct((E, F_IN), jnp.float32),
        mesh=mesh,
        compiler_params=pltpu.CompilerParams(use_tc_tiling_on_sc=False, needs_layout_passes=False),
        scratch_types=[
            pltpu.VMEM((CHUNK,), jnp.int32),
            pltpu.VMEM((CHUNK, F_IN), jnp.float32),
            pltpu.SemaphoreType.DMA,
        ],
    )
    def _gather_rows(x_hbm, idx_hbm, out_hbm, idx_v, rows_v, sem):
        w = _worker_id()

        def body(j, carry):
            cid = w + NW * j

            @pl.when(cid < NCHUNKS)
            def _():
                base = cid * CHUNK
                pltpu.sync_copy(idx_hbm.at[pl.ds(base, CHUNK)], idx_v)
                pltpu.async_copy(x_hbm.at[idx_v], rows_v, sem).wait()
                pltpu.sync_copy(rows_v, out_hbm.at[pl.ds(base, CHUNK)])

            return carry

        lax.fori_loop(0, CPW, body, 0)

    @functools.partial(
        pl.kernel,
        out_type=(
            jax.ShapeDtypeStruct((NC, N, HID), jnp.float32),
            jax.ShapeDtypeStruct((NC, N, HID), jnp.float32),
        ),
        mesh=mesh,
        compiler_params=pltpu.CompilerParams(use_tc_tiling_on_sc=False, needs_layout_passes=False),
        scratch_types=[
            pltpu.VMEM((N,), jnp.float32),
            pltpu.VMEM((N,), jnp.float32),
            pltpu.VMEM((16,), jnp.float32),
            pltpu.VMEM((CHUNK,), jnp.int32),
            pltpu.VMEM((CHUNK,), jnp.int32),
            pltpu.VMEM((CHUNK, HID), jnp.float32),
            pltpu.VMEM((CHUNK, HID), jnp.float32),
            pltpu.VMEM_SHARED((N, HID), jnp.float32),
            pltpu.VMEM_SHARED((N, HID), jnp.float32),
            pltpu.SemaphoreType.DMA,
        ],
    )
    def _gat_fused(xw_hbm, as_hbm, ad_hbm, src_hbm, dst_hbm, c_hbm, zeros_hbm,
                   numo_hbm, deno_hbm,
                   a_s_v, a_d_v, c_v, idxs_v, idxd_v, rows_v, prep_v,
                   num_acc, den_acc, sem):
        c = lax.axis_index("c")
        s = lax.axis_index("s")
        w = s * NC + c
        pltpu.sync_copy(zeros_hbm.at[pl.ds(s * RPS, RPS)], num_acc.at[pl.ds(s * RPS, RPS)])
        pltpu.sync_copy(zeros_hbm.at[pl.ds(s * RPS, RPS)], den_acc.at[pl.ds(s * RPS, RPS)])
        pltpu.sync_copy(as_hbm, a_s_v)
        pltpu.sync_copy(ad_hbm, a_d_v)
        pltpu.sync_copy(c_hbm, c_v)
        plsc.subcore_barrier()
        cvec = c_v[...]

        def body(j, carry):
            cid = w + NW * j

            @pl.when(cid < NCHUNKS)
            def _():
                base = cid * CHUNK
                pltpu.sync_copy(src_hbm.at[pl.ds(base, CHUNK)], idxs_v)
                pltpu.sync_copy(dst_hbm.at[pl.ds(base, CHUNK)], idxd_v)
                pltpu.async_copy(xw_hbm.at[idxs_v], rows_v, sem).wait()
                for k in range(CHUNK // 16):
                    si = idxs_v[pl.ds(k * 16, 16)]
                    di = idxd_v[pl.ds(k * 16, 16)]
                    z = plsc.load_gather(a_s_v, [si]) + plsc.load_gather(a_d_v, [di])
                    p_vec = jnp.exp(jnp.maximum(z, 0.2 * z) - cvec)
                    for e16 in range(16):
                        pv = p_vec[e16]
                        row = k * 16 + e16
                        rows_v[row, :] = rows_v[row, :] * pv
                        prep_v[row, :] = jnp.full((16,), pv, jnp.float32)
                pltpu.sync_copy(rows_v, num_acc.at[idxd_v], add=True)
                pltpu.sync_copy(prep_v, den_acc.at[idxd_v], add=True)

            return carry

        lax.fori_loop(0, CPW, body, 0)
        plsc.subcore_barrier()
        pltpu.sync_copy(num_acc.at[pl.ds(s * RPS, RPS)],
                        numo_hbm.at[c, pl.ds(s * RPS, RPS)])
        pltpu.sync_copy(den_acc.at[pl.ds(s * RPS, RPS)],
                        deno_hbm.at[c, pl.ds(s * RPS, RPS)])

    @functools.partial(
        pl.kernel,
        out_type=jax.ShapeDtypeStruct((NC, N, F_IN), jnp.float32),
        mesh=mesh,
        compiler_params=pltpu.CompilerParams(use_tc_tiling_on_sc=False, needs_layout_passes=False),
        scratch_types=[
            pltpu.VMEM((CHUNK,), jnp.int32),
            pltpu.VMEM((CHUNK, F_IN), jnp.float32),
            pltpu.VMEM_SHARED((N, F_IN), jnp.float32),
            pltpu.SemaphoreType.DMA,
        ],
    )
    def _scatter_add1(zeros_hbm, dst_hbm, rows_hbm, out_hbm, idx_v, rows_v, acc, sem):
        c = lax.axis_index("c")
        s = lax.axis_index("s")
        w = s * NC + c
        pltpu.sync_copy(zeros_hbm.at[pl.ds(s * RPS, RPS)], acc.at[pl.ds(s * RPS, RPS)])
        plsc.subcore_barrier()

        def body(j, carry):
            cid = w + NW * j

            @pl.when(cid < NCHUNKS)
            def _():
                base = cid * CHUNK
                pltpu.sync_copy(dst_hbm.at[pl.ds(base, CHUNK)], idx_v)
                pltpu.sync_copy(rows_hbm.at[pl.ds(base, CHUNK)], rows_v)
                pltpu.sync_copy(rows_v, acc.at[idx_v], add=True)

            return carry

        lax.fori_loop(0, CPW, body, 0)
        plsc.subcore_barrier()
        pltpu.sync_copy(acc.at[pl.ds(s * RPS, RPS)],
                        out_hbm.at[c, pl.ds(s * RPS, RPS)])

    return _gather_rows, _gat_fused, _scatter_add1


BE = 2000


def _mlp_body(ea_ref, xs_ref, w1_ref, b1_ref, w2_ref, b2_ref, w3_ref, b3_ref,
              q_ref, p_ref, out_ref):
    f32 = jnp.float32
    h1 = jnp.maximum(
        jnp.dot(ea_ref[...], w1_ref[...], preferred_element_type=f32) + b1_ref[...], 0.0)
    h2 = jnp.maximum(
        jnp.dot(h1, w2_ref[...], preferred_element_type=f32) + b2_ref[...], 0.0)
    w = jnp.dot(h2, w3_ref[...], preferred_element_type=f32) + b3_ref[...]
    xr = jnp.dot(xs_ref[...], q_ref[...], preferred_element_type=f32)
    out_ref[...] = jnp.dot(w * xr, p_ref[...], preferred_element_type=f32)


_edge_mlp = pl.pallas_call(
    _mlp_body,
    grid=(E // BE,),
    in_specs=[
        pl.BlockSpec((BE, F_EDGE), lambda i: (i, 0)),
        pl.BlockSpec((BE, F_IN), lambda i: (i, 0)),
        pl.BlockSpec((F_EDGE, 128), lambda i: (0, 0)),
        pl.BlockSpec((1, 128), lambda i: (0, 0)),
        pl.BlockSpec((128, 64), lambda i: (0, 0)),
        pl.BlockSpec((1, 64), lambda i: (0, 0)),
        pl.BlockSpec((64, F_IN * HID), lambda i: (0, 0)),
        pl.BlockSpec((1, F_IN * HID), lambda i: (0, 0)),
        pl.BlockSpec((F_IN, F_IN * HID), lambda i: (0, 0)),
        pl.BlockSpec((F_IN * HID, HID), lambda i: (0, 0)),
    ],
    out_specs=pl.BlockSpec((BE, HID), lambda i: (i, 0)),
    out_shape=jax.ShapeDtypeStruct((E, HID), jnp.float32),
)


def _node_body(x_ref, a0_ref, a1_ref, wr_ref, bc_ref, wg_ref, asv_ref, adv_ref,
               xw_ref, as_ref, ad_ref, c_ref, num0_ref, den0_ref):
    f32 = jnp.float32
    agg = a0_ref[...] + a1_ref[...]
    x1 = jnp.maximum(
        jnp.dot(x_ref[...], wr_ref[...], preferred_element_type=f32) + agg + bc_ref[...],
        0.0)
    xw = jnp.dot(x1, wg_ref[...], preferred_element_type=f32)
    a_s = jnp.sum(xw * asv_ref[...], axis=1, keepdims=True)
    a_d = jnp.sum(xw * adv_ref[...], axis=1, keepdims=True)
    cval = jnp.max(a_s) + jnp.max(a_d)
    z = a_s + a_d
    ps = jnp.exp(jnp.where(z >= 0, z, 0.2 * z) - cval)
    xw_ref[...] = xw
    as_ref[...] = a_s
    ad_ref[...] = a_d
    c_ref[...] = jnp.reshape(cval, (1, 1))
    num0_ref[...] = ps * xw
    den0_ref[...] = ps


_node_stage = pl.pallas_call(
    _node_body,
    out_shape=(
        jax.ShapeDtypeStruct((N, HID), jnp.float32),
        jax.ShapeDtypeStruct((N, 1), jnp.float32),
        jax.ShapeDtypeStruct((N, 1), jnp.float32),
        jax.ShapeDtypeStruct((1, 1), jnp.float32),
        jax.ShapeDtypeStruct((N, HID), jnp.float32),
        jax.ShapeDtypeStruct((N, 1), jnp.float32),
    ),
)


def _final_body(n0_ref, n1_ref, n2_ref, d0_ref, d1_ref, d2_ref, bg_ref, wf_ref,
                bf_ref, out_ref):
    f32 = jnp.float32
    num = n0_ref[...] + n1_ref[...] + n2_ref[...]
    den = d0_ref[...] + d1_ref[...] + d2_ref[...]
    out_g = num / den + bg_ref[...]
    x2 = jnp.maximum(out_g, 0.0)
    out_ref[...] = jnp.maximum(
        jnp.dot(x2, wf_ref[...], preferred_element_type=f32) + bf_ref[...], 0.0)


_final_stage = pl.pallas_call(
    _final_body,
    out_shape=jax.ShapeDtypeStruct((N, OUT), jnp.float32),
)


def kernel(x, edge_index, edge_attr, batch, W1, b1, W2, b2, W3, b3,
           W_root, b_conv, W_gat, att_src, att_dst, b_gat, W_fc1, b_fc1):
    _gather_rows, _gat_fused, _scatter_add1 = _sc_kernels()
    src = edge_index[0]
    dst = edge_index[1]
    zeros_n = jnp.zeros((N, F_IN), jnp.float32)
    jcol = jnp.arange(F_IN * HID)
    qmat = (jnp.arange(F_IN)[:, None] == (jcol[None, :] // HID)).astype(jnp.float32)
    pmat = ((jcol[:, None] % HID) == jnp.arange(HID)[None, :]).astype(jnp.float32)

    xs = _gather_rows(x, src)
    msg = _edge_mlp(edge_attr, xs, W1, b1.reshape(1, -1), W2, b2.reshape(1, -1),
                    W3, b3.reshape(1, -1), qmat, pmat)
    aggp = _scatter_add1(zeros_n, dst, msg)
    xw, a_s, a_d, cvl, num0, den0 = _node_stage(
        x, aggp[0], aggp[1], W_root, b_conv.reshape(1, -1), W_gat,
        att_src.reshape(1, -1), att_dst.reshape(1, -1))
    c_rep = jnp.broadcast_to(cvl.reshape(1), (16,))
    nump, denp = _gat_fused(xw, a_s.reshape(N), a_d.reshape(N), src, dst,
                            c_rep, zeros_n)
    out = _final_stage(num0, nump[0], nump[1], den0, denp[0], denp[1],
                       b_gat.reshape(1, -1), W_fc1, b_fc1.reshape(1, -1))
    return out


# speedup vs baseline: 10.8730x; 1.0613x over previous
import functools

import jax
import jax.numpy as jnp
from jax import lax
from jax.experimental import pallas as pl
from jax.experimental.pallas import tpu as pltpu
from jax.experimental.pallas import tpu_sc as plsc

N = 10000
E = 160000
F_IN = 16
F_EDGE = 16
HID = 16
OUT = 64

NC = 2
NS = 16
NW = NC * NS
CHUNK = 128
NCHUNKS = E // CHUNK
CPW = -(-NCHUNKS // NW)
RPS = N // NS

def _worker_id():
    return lax.axis_index("s") * NC + lax.axis_index("c")


@functools.cache
def _sc_kernels():
    mesh = plsc.VectorSubcoreMesh(
        core_axis_name="c", subcore_axis_name="s", num_cores=NC, num_subcores=NS)

    @functools.partial(
        pl.kernel,
        out_type=jax.ShapeDtypeStruct((E, F_IN), jnp.float32),
        mesh=mesh,
        compiler_params=pltpu.CompilerParams(use_tc_tiling_on_sc=False, needs_layout_passes=False),
        scratch_types=[
            pltpu.VMEM((CHUNK,), jnp.int32),
            pltpu.VMEM((CHUNK, F_IN), jnp.float32),
            pltpu.SemaphoreType.DMA,
        ],
    )
    def _gather_rows(x_hbm, idx_hbm, out_hbm, idx_v, rows_v, sem):
        w = _worker_id()

        def body(j, carry):
            cid = w + NW * j

            @pl.when(cid < NCHUNKS)
            def _():
                base = cid * CHUNK
                pltpu.sync_copy(idx_hbm.at[pl.ds(base, CHUNK)], idx_v)
                pltpu.async_copy(x_hbm.at[idx_v], rows_v, sem).wait()
                pltpu.sync_copy(rows_v, out_hbm.at[pl.ds(base, CHUNK)])

            return carry

        lax.fori_loop(0, CPW, body, 0)

    @functools.partial(
        pl.kernel,
        out_type=(
            jax.ShapeDtypeStruct((NC, N, HID), jnp.float32),
            jax.ShapeDtypeStruct((NC, N, HID), jnp.float32),
        ),
        mesh=mesh,
        compiler_params=pltpu.CompilerParams(use_tc_tiling_on_sc=False, needs_layout_passes=False),
        scratch_types=[
            pltpu.VMEM((N,), jnp.float32),
            pltpu.VMEM((N,), jnp.float32),
            pltpu.VMEM((16,), jnp.float32),
            pltpu.VMEM((CHUNK,), jnp.int32),
            pltpu.VMEM((CHUNK,), jnp.int32),
            pltpu.VMEM((CHUNK, HID), jnp.float32),
            pltpu.VMEM((CHUNK, HID), jnp.float32),
            pltpu.VMEM_SHARED((N, HID), jnp.float32),
            pltpu.VMEM_SHARED((N, HID), jnp.float32),
            pltpu.SemaphoreType.DMA,
        ],
    )
    def _gat_fused(xw_hbm, as_hbm, ad_hbm, src_hbm, dst_hbm, c_hbm, zeros_hbm,
                   numo_hbm, deno_hbm,
                   a_s_v, a_d_v, c_v, idxs_v, idxd_v, rows_v, prep_v,
                   num_acc, den_acc, sem):
        c = lax.axis_index("c")
        s = lax.axis_index("s")
        w = s * NC + c
        pltpu.sync_copy(zeros_hbm.at[pl.ds(s * RPS, RPS)], num_acc.at[pl.ds(s * RPS, RPS)])
        pltpu.sync_copy(zeros_hbm.at[pl.ds(s * RPS, RPS)], den_acc.at[pl.ds(s * RPS, RPS)])
        pltpu.sync_copy(as_hbm, a_s_v)
        pltpu.sync_copy(ad_hbm, a_d_v)
        pltpu.sync_copy(c_hbm, c_v)
        plsc.subcore_barrier()
        cvec = c_v[...]

        def body(j, carry):
            cid = w + NW * j

            @pl.when(cid < NCHUNKS)
            def _():
                base = cid * CHUNK
                pltpu.sync_copy(src_hbm.at[pl.ds(base, CHUNK)], idxs_v)
                pltpu.sync_copy(dst_hbm.at[pl.ds(base, CHUNK)], idxd_v)
                pltpu.async_copy(xw_hbm.at[idxs_v], rows_v, sem).wait()
                for k in range(CHUNK // 16):
                    si = idxs_v[pl.ds(k * 16, 16)]
                    di = idxd_v[pl.ds(k * 16, 16)]
                    z = plsc.load_gather(a_s_v, [si]) + plsc.load_gather(a_d_v, [di])
                    p_vec = jnp.exp(jnp.maximum(z, 0.2 * z) - cvec)
                    for e16 in range(16):
                        pv = p_vec[e16]
                        row = k * 16 + e16
                        rows_v[row, :] = rows_v[row, :] * pv
                        prep_v[row, :] = jnp.full((16,), pv, jnp.float32)
                pltpu.sync_copy(rows_v, num_acc.at[idxd_v], add=True)
                pltpu.sync_copy(prep_v, den_acc.at[idxd_v], add=True)

            return carry

        lax.fori_loop(0, CPW, body, 0)
        plsc.subcore_barrier()
        pltpu.sync_copy(num_acc.at[pl.ds(s * RPS, RPS)],
                        numo_hbm.at[c, pl.ds(s * RPS, RPS)])
        pltpu.sync_copy(den_acc.at[pl.ds(s * RPS, RPS)],
                        deno_hbm.at[c, pl.ds(s * RPS, RPS)])

    @functools.partial(
        pl.kernel,
        out_type=jax.ShapeDtypeStruct((NC, N, F_IN), jnp.float32),
        mesh=mesh,
        compiler_params=pltpu.CompilerParams(use_tc_tiling_on_sc=False, needs_layout_passes=False),
        scratch_types=[
            pltpu.VMEM((CHUNK,), jnp.int32),
            pltpu.VMEM((CHUNK, F_IN), jnp.float32),
            pltpu.VMEM_SHARED((N, F_IN), jnp.float32),
            pltpu.SemaphoreType.DMA,
        ],
    )
    def _scatter_add1(zeros_hbm, dst_hbm, rows_hbm, out_hbm, idx_v, rows_v, acc, sem):
        c = lax.axis_index("c")
        s = lax.axis_index("s")
        w = s * NC + c
        pltpu.sync_copy(zeros_hbm.at[pl.ds(s * RPS, RPS)], acc.at[pl.ds(s * RPS, RPS)])
        plsc.subcore_barrier()

        def body(j, carry):
            cid = w + NW * j

            @pl.when(cid < NCHUNKS)
            def _():
                base = cid * CHUNK
                pltpu.sync_copy(dst_hbm.at[pl.ds(base, CHUNK)], idx_v)
                pltpu.sync_copy(rows_hbm.at[pl.ds(base, CHUNK)], rows_v)
                pltpu.sync_copy(rows_v, acc.at[idx_v], add=True)

            return carry

        lax.fori_loop(0, CPW, body, 0)
        plsc.subcore_barrier()
        pltpu.sync_copy(acc.at[pl.ds(s * RPS, RPS)],
                        out_hbm.at[c, pl.ds(s * RPS, RPS)])

    return _gather_rows, _gat_fused, _scatter_add1


BE = 3200


def _mlp_body(eat_ref, xs_ref, w1_ref, b1_ref, w2_ref, b2_ref, w3_ref, b3_ref,
              q_ref, p_ref, out_ref):
    f32 = jnp.float32
    h1 = jnp.maximum(
        lax.dot_general(eat_ref[...], w1_ref[...], (((0,), (0,)), ((), ())),
                        preferred_element_type=f32) + b1_ref[...], 0.0)
    h2 = jnp.maximum(
        jnp.dot(h1, w2_ref[...], preferred_element_type=f32) + b2_ref[...], 0.0)
    w = jnp.dot(h2, w3_ref[...], preferred_element_type=f32) + b3_ref[...]
    xr = jnp.dot(xs_ref[...], q_ref[...], preferred_element_type=f32)
    out_ref[...] = jnp.dot(w * xr, p_ref[...], preferred_element_type=f32)


_edge_mlp = pl.pallas_call(
    _mlp_body,
    grid=(E // BE,),
    in_specs=[
        pl.BlockSpec((F_EDGE, BE), lambda i: (0, i)),
        pl.BlockSpec((BE, F_IN), lambda i: (i, 0)),
        pl.BlockSpec((F_EDGE, 128), lambda i: (0, 0)),
        pl.BlockSpec((1, 128), lambda i: (0, 0)),
        pl.BlockSpec((128, 64), lambda i: (0, 0)),
        pl.BlockSpec((1, 64), lambda i: (0, 0)),
        pl.BlockSpec((64, F_IN * HID), lambda i: (0, 0)),
        pl.BlockSpec((1, F_IN * HID), lambda i: (0, 0)),
        pl.BlockSpec((F_IN, F_IN * HID), lambda i: (0, 0)),
        pl.BlockSpec((F_IN * HID, HID), lambda i: (0, 0)),
    ],
    out_specs=pl.BlockSpec((BE, HID), lambda i: (i, 0)),
    out_shape=jax.ShapeDtypeStruct((E, HID), jnp.float32),
)


def _node_body(x_ref, a0_ref, a1_ref, wr_ref, bc_ref, wg_ref, asv_ref, adv_ref,
               xw_ref, as_ref, ad_ref, c_ref, num0_ref, den0_ref):
    f32 = jnp.float32
    agg = a0_ref[...] + a1_ref[...]
    x1 = jnp.maximum(
        jnp.dot(x_ref[...], wr_ref[...], preferred_element_type=f32) + agg + bc_ref[...],
        0.0)
    xw = jnp.dot(x1, wg_ref[...], preferred_element_type=f32)
    a_s = jnp.sum(xw * asv_ref[...], axis=1, keepdims=True)
    a_d = jnp.sum(xw * adv_ref[...], axis=1, keepdims=True)
    cval = jnp.max(a_s) + jnp.max(a_d)
    z = a_s + a_d
    ps = jnp.exp(jnp.where(z >= 0, z, 0.2 * z) - cval)
    xw_ref[...] = xw
    as_ref[...] = a_s
    ad_ref[...] = a_d
    c_ref[...] = jnp.reshape(cval, (1, 1))
    num0_ref[...] = ps * xw
    den0_ref[...] = ps


_node_stage = pl.pallas_call(
    _node_body,
    out_shape=(
        jax.ShapeDtypeStruct((N, HID), jnp.float32),
        jax.ShapeDtypeStruct((N, 1), jnp.float32),
        jax.ShapeDtypeStruct((N, 1), jnp.float32),
        jax.ShapeDtypeStruct((1, 1), jnp.float32),
        jax.ShapeDtypeStruct((N, HID), jnp.float32),
        jax.ShapeDtypeStruct((N, 1), jnp.float32),
    ),
)


def _final_body(n0_ref, n1_ref, n2_ref, d0_ref, d1_ref, d2_ref, bg_ref, wf_ref,
                bf_ref, out_ref):
    f32 = jnp.float32
    num = n0_ref[...] + n1_ref[...] + n2_ref[...]
    den = d0_ref[...] + d1_ref[...] + d2_ref[...]
    out_g = num / den + bg_ref[...]
    x2 = jnp.maximum(out_g, 0.0)
    out_ref[...] = jnp.maximum(
        jnp.dot(x2, wf_ref[...], preferred_element_type=f32) + bf_ref[...], 0.0)


_final_stage = pl.pallas_call(
    _final_body,
    out_shape=jax.ShapeDtypeStruct((N, OUT), jnp.float32),
)


def kernel(x, edge_index, edge_attr, batch, W1, b1, W2, b2, W3, b3,
           W_root, b_conv, W_gat, att_src, att_dst, b_gat, W_fc1, b_fc1):
    _gather_rows, _gat_fused, _scatter_add1 = _sc_kernels()
    src = edge_index[0]
    dst = edge_index[1]
    zeros_n = jnp.zeros((N, F_IN), jnp.float32)
    jcol = jnp.arange(F_IN * HID)
    qmat = (jnp.arange(F_IN)[:, None] == (jcol[None, :] // HID)).astype(jnp.float32)
    pmat = ((jcol[:, None] % HID) == jnp.arange(HID)[None, :]).astype(jnp.float32)

    xs = _gather_rows(x, src)
    msg = _edge_mlp(edge_attr.T, xs, W1, b1.reshape(1, -1),
                    W2, b2.reshape(1, -1), W3, b3.reshape(1, -1), qmat, pmat)
    aggp = _scatter_add1(zeros_n, dst, msg)
    xw, a_s, a_d, cvl, num0, den0 = _node_stage(
        x, aggp[0], aggp[1], W_root, b_conv.reshape(1, -1), W_gat,
        att_src.reshape(1, -1), att_dst.reshape(1, -1))
    c_rep = jnp.broadcast_to(cvl.reshape(1), (16,))
    nump, denp = _gat_fused(xw, a_s.reshape(N), a_d.reshape(N), src, dst,
                            c_rep, zeros_n)
    out = _final_stage(num0, nump[0], nump[1], den0, denp[0], denp[1],
                       b_gat.reshape(1, -1), W_fc1, b_fc1.reshape(1, -1))
    return out


# speedup vs baseline: 11.1922x; 1.0294x over previous
import functools

import jax
import jax.numpy as jnp
from jax import lax
from jax.experimental import pallas as pl
from jax.experimental.pallas import tpu as pltpu
from jax.experimental.pallas import tpu_sc as plsc

N = 10000
E = 160000
F_IN = 16
F_EDGE = 16
HID = 16
OUT = 64

NC = 2
NS = 16
NW = NC * NS
CHUNK = 128
NCHUNKS = E // CHUNK
CPW = -(-NCHUNKS // NW)
RPS = N // NS

def _worker_id():
    return lax.axis_index("s") * NC + lax.axis_index("c")


@functools.cache
def _sc_kernels():
    mesh = plsc.VectorSubcoreMesh(
        core_axis_name="c", subcore_axis_name="s", num_cores=NC, num_subcores=NS)

    @functools.partial(
        pl.kernel,
        out_type=jax.ShapeDtypeStruct((F_IN, E), jnp.float32),
        mesh=mesh,
        compiler_params=pltpu.CompilerParams(use_tc_tiling_on_sc=False, needs_layout_passes=False),
        scratch_types=[
            pltpu.VMEM((CHUNK,), jnp.int32),
            pltpu.VMEM((CHUNK, F_IN), jnp.float32),
            pltpu.VMEM((F_IN, CHUNK), jnp.float32),
            pltpu.SemaphoreType.DMA,
        ],
    )
    def _gather_rows(x_hbm, idx_hbm, out_hbm, idx_v, rows_v, buf_t, sem):
        w = _worker_id()
        lane = lax.iota(jnp.int32, 16)

        def body(j, carry):
            cid = w + NW * j

            @pl.when(cid < NCHUNKS)
            def _():
                base = cid * CHUNK
                pltpu.sync_copy(idx_hbm.at[pl.ds(base, CHUNK)], idx_v)
                pltpu.async_copy(x_hbm.at[idx_v], rows_v, sem).wait()
                for f in range(F_IN):
                    fv = jnp.full((16,), f, jnp.int32)
                    for g in range(CHUNK // 16):
                        col = plsc.load_gather(rows_v, [lane + g * 16, fv])
                        buf_t[f, pl.ds(g * 16, 16)] = col
                pltpu.sync_copy(buf_t, out_hbm.at[:, pl.ds(base, CHUNK)])

            return carry

        lax.fori_loop(0, CPW, body, 0)

    @functools.partial(
        pl.kernel,
        out_type=(
            jax.ShapeDtypeStruct((NC, N, HID), jnp.float32),
            jax.ShapeDtypeStruct((NC, N, HID), jnp.float32),
        ),
        mesh=mesh,
        compiler_params=pltpu.CompilerParams(use_tc_tiling_on_sc=False, needs_layout_passes=False),
        scratch_types=[
            pltpu.VMEM((N,), jnp.float32),
            pltpu.VMEM((N,), jnp.float32),
            pltpu.VMEM((16,), jnp.float32),
            pltpu.VMEM((CHUNK,), jnp.int32),
            pltpu.VMEM((CHUNK,), jnp.int32),
            pltpu.VMEM((CHUNK, HID), jnp.float32),
            pltpu.VMEM((CHUNK, HID), jnp.float32),
            pltpu.VMEM_SHARED((N, HID), jnp.float32),
            pltpu.VMEM_SHARED((N, HID), jnp.float32),
            pltpu.SemaphoreType.DMA,
        ],
    )
    def _gat_fused(xw_hbm, as_hbm, ad_hbm, src_hbm, dst_hbm, c_hbm, zeros_hbm,
                   numo_hbm, deno_hbm,
                   a_s_v, a_d_v, c_v, idxs_v, idxd_v, rows_v, prep_v,
                   num_acc, den_acc, sem):
        c = lax.axis_index("c")
        s = lax.axis_index("s")
        w = s * NC + c
        pltpu.sync_copy(zeros_hbm.at[pl.ds(s * RPS, RPS)], num_acc.at[pl.ds(s * RPS, RPS)])
        pltpu.sync_copy(zeros_hbm.at[pl.ds(s * RPS, RPS)], den_acc.at[pl.ds(s * RPS, RPS)])
        pltpu.sync_copy(as_hbm, a_s_v)
        pltpu.sync_copy(ad_hbm, a_d_v)
        pltpu.sync_copy(c_hbm, c_v)
        plsc.subcore_barrier()
        cvec = c_v[...]

        def body(j, carry):
            cid = w + NW * j

            @pl.when(cid < NCHUNKS)
            def _():
                base = cid * CHUNK
                pltpu.sync_copy(src_hbm.at[pl.ds(base, CHUNK)], idxs_v)
                pltpu.sync_copy(dst_hbm.at[pl.ds(base, CHUNK)], idxd_v)
                pltpu.async_copy(xw_hbm.at[idxs_v], rows_v, sem).wait()
                for k in range(CHUNK // 16):
                    si = idxs_v[pl.ds(k * 16, 16)]
                    di = idxd_v[pl.ds(k * 16, 16)]
                    z = plsc.load_gather(a_s_v, [si]) + plsc.load_gather(a_d_v, [di])
                    p_vec = jnp.exp(jnp.maximum(z, 0.2 * z) - cvec)
                    for e16 in range(16):
                        pv = p_vec[e16]
                        row = k * 16 + e16
                        rows_v[row, :] = rows_v[row, :] * pv
                        prep_v[row, :] = jnp.full((16,), pv, jnp.float32)
                pltpu.sync_copy(rows_v, num_acc.at[idxd_v], add=True)
                pltpu.sync_copy(prep_v, den_acc.at[idxd_v], add=True)

            return carry

        lax.fori_loop(0, CPW, body, 0)
        plsc.subcore_barrier()
        pltpu.sync_copy(num_acc.at[pl.ds(s * RPS, RPS)],
                        numo_hbm.at[c, pl.ds(s * RPS, RPS)])
        pltpu.sync_copy(den_acc.at[pl.ds(s * RPS, RPS)],
                        deno_hbm.at[c, pl.ds(s * RPS, RPS)])

    @functools.partial(
        pl.kernel,
        out_type=jax.ShapeDtypeStruct((NC, N, F_IN), jnp.float32),
        mesh=mesh,
        compiler_params=pltpu.CompilerParams(use_tc_tiling_on_sc=False, needs_layout_passes=False),
        scratch_types=[
            pltpu.VMEM((CHUNK,), jnp.int32),
            pltpu.VMEM((F_IN, CHUNK), jnp.float32),
            pltpu.VMEM((CHUNK, F_IN), jnp.float32),
            pltpu.VMEM_SHARED((N, F_IN), jnp.float32),
            pltpu.SemaphoreType.DMA,
        ],
    )
    def _scatter_add1(zeros_hbm, dst_hbm, rowst_hbm, out_hbm, idx_v, rowst_v,
                      rows_v, acc, sem):
        c = lax.axis_index("c")
        s = lax.axis_index("s")
        w = s * NC + c
        lane = lax.iota(jnp.int32, 16)
        pltpu.sync_copy(zeros_hbm.at[pl.ds(s * RPS, RPS)], acc.at[pl.ds(s * RPS, RPS)])
        plsc.subcore_barrier()

        def body(j, carry):
            cid = w + NW * j

            @pl.when(cid < NCHUNKS)
            def _():
                base = cid * CHUNK
                pltpu.sync_copy(dst_hbm.at[pl.ds(base, CHUNK)], idx_v)
                pltpu.sync_copy(rowst_hbm.at[:, pl.ds(base, CHUNK)], rowst_v)
                for e16 in range(CHUNK):
                    ev = jnp.full((16,), e16, jnp.int32)
                    rows_v[e16, :] = plsc.load_gather(rowst_v, [lane, ev])
                pltpu.sync_copy(rows_v, acc.at[idx_v], add=True)

            return carry

        lax.fori_loop(0, CPW, body, 0)
        plsc.subcore_barrier()
        pltpu.sync_copy(acc.at[pl.ds(s * RPS, RPS)],
                        out_hbm.at[c, pl.ds(s * RPS, RPS)])

    return _gather_rows, _gat_fused, _scatter_add1


BE = 3200


def _mlp_body(eat_ref, xst_ref, w1_ref, b1_ref, w2_ref, b2_ref, w3_ref, b3_ref,
              q_ref, p_ref, out_ref):
    f32 = jnp.float32
    dnum = (((0,), (0,)), ((), ()))
    h1t = jnp.maximum(
        lax.dot_general(w1_ref[...], eat_ref[...], dnum,
                        preferred_element_type=f32) + b1_ref[...], 0.0)
    h2t = jnp.maximum(
        lax.dot_general(w2_ref[...], h1t, dnum,
                        preferred_element_type=f32) + b2_ref[...], 0.0)
    wt = lax.dot_general(w3_ref[...], h2t, dnum,
                         preferred_element_type=f32) + b3_ref[...]
    xrt = lax.dot_general(q_ref[...], xst_ref[...], dnum,
                          preferred_element_type=f32)
    out_ref[...] = lax.dot_general(p_ref[...], wt * xrt, dnum,
                                   preferred_element_type=f32)


_edge_mlp = pl.pallas_call(
    _mlp_body,
    grid=(E // BE,),
    in_specs=[
        pl.BlockSpec((F_EDGE, BE), lambda i: (0, i)),
        pl.BlockSpec((F_IN, BE), lambda i: (0, i)),
        pl.BlockSpec((F_EDGE, 128), lambda i: (0, 0)),
        pl.BlockSpec((128, 1), lambda i: (0, 0)),
        pl.BlockSpec((128, 64), lambda i: (0, 0)),
        pl.BlockSpec((64, 1), lambda i: (0, 0)),
        pl.BlockSpec((64, F_IN * HID), lambda i: (0, 0)),
        pl.BlockSpec((F_IN * HID, 1), lambda i: (0, 0)),
        pl.BlockSpec((F_IN, F_IN * HID), lambda i: (0, 0)),
        pl.BlockSpec((F_IN * HID, HID), lambda i: (0, 0)),
    ],
    out_specs=pl.BlockSpec((F_IN, BE), lambda i: (0, i)),
    out_shape=jax.ShapeDtypeStruct((F_IN, E), jnp.float32),
)


def _node_body(x_ref, a0_ref, a1_ref, wr_ref, bc_ref, wg_ref, asv_ref, adv_ref,
               xw_ref, as_ref, ad_ref, c_ref, num0_ref, den0_ref):
    f32 = jnp.float32
    agg = a0_ref[...] + a1_ref[...]
    x1 = jnp.maximum(
        jnp.dot(x_ref[...], wr_ref[...], preferred_element_type=f32) + agg + bc_ref[...],
        0.0)
    xw = jnp.dot(x1, wg_ref[...], preferred_element_type=f32)
    a_s = jnp.sum(xw * asv_ref[...], axis=1, keepdims=True)
    a_d = jnp.sum(xw * adv_ref[...], axis=1, keepdims=True)
    cval = jnp.max(a_s) + jnp.max(a_d)
    z = a_s + a_d
    ps = jnp.exp(jnp.where(z >= 0, z, 0.2 * z) - cval)
    xw_ref[...] = xw
    as_ref[...] = a_s
    ad_ref[...] = a_d
    c_ref[...] = jnp.reshape(cval, (1, 1))
    num0_ref[...] = ps * xw
    den0_ref[...] = ps


_node_stage = pl.pallas_call(
    _node_body,
    out_shape=(
        jax.ShapeDtypeStruct((N, HID), jnp.float32),
        jax.ShapeDtypeStruct((N, 1), jnp.float32),
        jax.ShapeDtypeStruct((N, 1), jnp.float32),
        jax.ShapeDtypeStruct((1, 1), jnp.float32),
        jax.ShapeDtypeStruct((N, HID), jnp.float32),
        jax.ShapeDtypeStruct((N, 1), jnp.float32),
    ),
)


def _final_body(n0_ref, n1_ref, n2_ref, d0_ref, d1_ref, d2_ref, bg_ref, wf_ref,
                bf_ref, out_ref):
    f32 = jnp.float32
    num = n0_ref[...] + n1_ref[...] + n2_ref[...]
    den = d0_ref[...] + d1_ref[...] + d2_ref[...]
    out_g = num / den + bg_ref[...]
    x2 = jnp.maximum(out_g, 0.0)
    out_ref[...] = jnp.maximum(
        jnp.dot(x2, wf_ref[...], preferred_element_type=f32) + bf_ref[...], 0.0)


_final_stage = pl.pallas_call(
    _final_body,
    out_shape=jax.ShapeDtypeStruct((N, OUT), jnp.float32),
)


def kernel(x, edge_index, edge_attr, batch, W1, b1, W2, b2, W3, b3,
           W_root, b_conv, W_gat, att_src, att_dst, b_gat, W_fc1, b_fc1):
    _gather_rows, _gat_fused, _scatter_add1 = _sc_kernels()
    src = edge_index[0]
    dst = edge_index[1]
    zeros_n = jnp.zeros((N, F_IN), jnp.float32)
    jcol = jnp.arange(F_IN * HID)
    qmat = (jnp.arange(F_IN)[:, None] == (jcol[None, :] // HID)).astype(jnp.float32)
    pmat = ((jcol[:, None] % HID) == jnp.arange(HID)[None, :]).astype(jnp.float32)

    xst = _gather_rows(x, src)
    msgt = _edge_mlp(edge_attr.T, xst, W1, b1.reshape(-1, 1),
                     W2, b2.reshape(-1, 1), W3, b3.reshape(-1, 1), qmat, pmat)
    aggp = _scatter_add1(zeros_n, dst, msgt)
    xw, a_s, a_d, cvl, num0, den0 = _node_stage(
        x, aggp[0], aggp[1], W_root, b_conv.reshape(1, -1), W_gat,
        att_src.reshape(1, -1), att_dst.reshape(1, -1))
    c_rep = jnp.broadcast_to(cvl.reshape(1), (16,))
    nump, denp = _gat_fused(xw, a_s.reshape(N), a_d.reshape(N), src, dst,
                            c_rep, zeros_n)
    out = _final_stage(num0, nump[0], nump[1], den0, denp[0], denp[1],
                       b_gat.reshape(1, -1), W_fc1, b_fc1.reshape(1, -1))
    return out


# speedup vs baseline: 11.9067x; 1.0638x over previous
import functools

import jax
import jax.numpy as jnp
from jax import lax
from jax.experimental import pallas as pl
from jax.experimental.pallas import tpu as pltpu
from jax.experimental.pallas import tpu_sc as plsc

N = 10000
E = 160000
F_IN = 16
F_EDGE = 16
HID = 16
OUT = 64

NC = 2
NS = 16
NW = NC * NS
CHUNK = 128
NCHUNKS = E // CHUNK
CPW = -(-NCHUNKS // NW)
RPS = N // NS

def _worker_id():
    return lax.axis_index("s") * NC + lax.axis_index("c")


@functools.cache
def _sc_kernels():
    mesh = plsc.VectorSubcoreMesh(
        core_axis_name="c", subcore_axis_name="s", num_cores=NC, num_subcores=NS)

    WAY = 4

    @functools.partial(
        pl.kernel,
        out_type=jax.ShapeDtypeStruct((F_IN, E), jnp.float32),
        mesh=mesh,
        compiler_params=pltpu.CompilerParams(use_tc_tiling_on_sc=False, needs_layout_passes=False),
        scratch_types=(
            [pltpu.VMEM((CHUNK,), jnp.int32) for _ in range(WAY)]
            + [pltpu.VMEM((CHUNK, F_IN), jnp.float32) for _ in range(WAY)]
            + [pltpu.VMEM((F_IN, CHUNK), jnp.float32) for _ in range(WAY)]
            + [pltpu.SemaphoreType.DMA((3 * WAY,))]
        ),
    )
    def _gather_rows(x_hbm, idx_hbm, out_hbm, *scr):
        idx_v = scr[0:WAY]
        rows_v = scr[WAY:2 * WAY]
        buf_t = scr[2 * WAY:3 * WAY]
        sems = scr[3 * WAY]
        w = _worker_id()
        lane = lax.iota(jnp.int32, 16)

        def body(j, carry):
            cids = [w + NW * (WAY * j + k) for k in range(WAY)]
            bases = [cid * CHUNK for cid in cids]
            for k in range(WAY):
                @pl.when(cids[k] < NCHUNKS)
                def _(k=k):
                    pltpu.async_copy(idx_hbm.at[pl.ds(bases[k], CHUNK)],
                                     idx_v[k], sems.at[k])
            for k in range(WAY):
                @pl.when(cids[k] < NCHUNKS)
                def _(k=k):
                    pltpu.make_async_copy(idx_hbm.at[pl.ds(bases[k], CHUNK)],
                                          idx_v[k], sems.at[k]).wait()
                    pltpu.async_copy(x_hbm.at[idx_v[k]], rows_v[k],
                                     sems.at[WAY + k])
            for k in range(WAY):
                @pl.when(cids[k] < NCHUNKS)
                def _(k=k):
                    pltpu.make_async_copy(x_hbm.at[idx_v[k]], rows_v[k],
                                          sems.at[WAY + k]).wait()
                    for f in range(F_IN):
                        fv = jnp.full((16,), f, jnp.int32)
                        for g in range(CHUNK // 16):
                            col = plsc.load_gather(rows_v[k], [lane + g * 16, fv])
                            buf_t[k][f, pl.ds(g * 16, 16)] = col
                    pltpu.async_copy(buf_t[k],
                                     out_hbm.at[:, pl.ds(bases[k], CHUNK)],
                                     sems.at[2 * WAY + k])
            for k in range(WAY):
                @pl.when(cids[k] < NCHUNKS)
                def _(k=k):
                    pltpu.make_async_copy(buf_t[k],
                                          out_hbm.at[:, pl.ds(bases[k], CHUNK)],
                                          sems.at[2 * WAY + k]).wait()
            return carry

        lax.fori_loop(0, CPW // WAY, body, 0)

    @functools.partial(
        pl.kernel,
        out_type=(
            jax.ShapeDtypeStruct((NC, N, HID), jnp.float32),
            jax.ShapeDtypeStruct((NC, N, HID), jnp.float32),
        ),
        mesh=mesh,
        compiler_params=pltpu.CompilerParams(use_tc_tiling_on_sc=False, needs_layout_passes=False),
        scratch_types=[
            pltpu.VMEM((N,), jnp.float32),
            pltpu.VMEM((N,), jnp.float32),
            pltpu.VMEM((16,), jnp.float32),
            pltpu.VMEM((CHUNK,), jnp.int32),
            pltpu.VMEM((CHUNK,), jnp.int32),
            pltpu.VMEM((CHUNK, HID), jnp.float32),
            pltpu.VMEM((CHUNK, HID), jnp.float32),
            pltpu.VMEM_SHARED((N, HID), jnp.float32),
            pltpu.VMEM_SHARED((N, HID), jnp.float32),
            pltpu.SemaphoreType.DMA,
        ],
    )
    def _gat_fused(xw_hbm, as_hbm, ad_hbm, src_hbm, dst_hbm, c_hbm, zeros_hbm,
                   numo_hbm, deno_hbm,
                   a_s_v, a_d_v, c_v, idxs_v, idxd_v, rows_v, prep_v,
                   num_acc, den_acc, sem):
        c = lax.axis_index("c")
        s = lax.axis_index("s")
        w = s * NC + c
        pltpu.sync_copy(zeros_hbm.at[pl.ds(s * RPS, RPS)], num_acc.at[pl.ds(s * RPS, RPS)])
        pltpu.sync_copy(zeros_hbm.at[pl.ds(s * RPS, RPS)], den_acc.at[pl.ds(s * RPS, RPS)])
        pltpu.sync_copy(as_hbm, a_s_v)
        pltpu.sync_copy(ad_hbm, a_d_v)
        pltpu.sync_copy(c_hbm, c_v)
        plsc.subcore_barrier()
        cvec = c_v[...]

        def body(j, carry):
            cid = w + NW * j

            @pl.when(cid < NCHUNKS)
            def _():
                base = cid * CHUNK
                pltpu.sync_copy(src_hbm.at[pl.ds(base, CHUNK)], idxs_v)
                pltpu.sync_copy(dst_hbm.at[pl.ds(base, CHUNK)], idxd_v)
                pltpu.async_copy(xw_hbm.at[idxs_v], rows_v, sem).wait()
                for k in range(CHUNK // 16):
                    si = idxs_v[pl.ds(k * 16, 16)]
                    di = idxd_v[pl.ds(k * 16, 16)]
                    z = plsc.load_gather(a_s_v, [si]) + plsc.load_gather(a_d_v, [di])
                    p_vec = jnp.exp(jnp.maximum(z, 0.2 * z) - cvec)
                    for e16 in range(16):
                        pv = p_vec[e16]
                        row = k * 16 + e16
                        rows_v[row, :] = rows_v[row, :] * pv
                        prep_v[row, :] = jnp.full((16,), pv, jnp.float32)
                pltpu.sync_copy(rows_v, num_acc.at[idxd_v], add=True)
                pltpu.sync_copy(prep_v, den_acc.at[idxd_v], add=True)

            return carry

        lax.fori_loop(0, CPW, body, 0)
        plsc.subcore_barrier()
        pltpu.sync_copy(num_acc.at[pl.ds(s * RPS, RPS)],
                        numo_hbm.at[c, pl.ds(s * RPS, RPS)])
        pltpu.sync_copy(den_acc.at[pl.ds(s * RPS, RPS)],
                        deno_hbm.at[c, pl.ds(s * RPS, RPS)])

    @functools.partial(
        pl.kernel,
        out_type=jax.ShapeDtypeStruct((NC, N, F_IN), jnp.float32),
        mesh=mesh,
        compiler_params=pltpu.CompilerParams(use_tc_tiling_on_sc=False, needs_layout_passes=False),
        scratch_types=[
            pltpu.VMEM((CHUNK,), jnp.int32),
            pltpu.VMEM((F_IN, CHUNK), jnp.float32),
            pltpu.VMEM((CHUNK, F_IN), jnp.float32),
            pltpu.VMEM_SHARED((N, F_IN), jnp.float32),
            pltpu.SemaphoreType.DMA,
        ],
    )
    def _scatter_add1(zeros_hbm, dst_hbm, rowst_hbm, out_hbm, idx_v, rowst_v,
                      rows_v, acc, sem):
        c = lax.axis_index("c")
        s = lax.axis_index("s")
        w = s * NC + c
        lane = lax.iota(jnp.int32, 16)
        pltpu.sync_copy(zeros_hbm.at[pl.ds(s * RPS, RPS)], acc.at[pl.ds(s * RPS, RPS)])
        plsc.subcore_barrier()

        def body(j, carry):
            cid = w + NW * j

            @pl.when(cid < NCHUNKS)
            def _():
                base = cid * CHUNK
                pltpu.sync_copy(dst_hbm.at[pl.ds(base, CHUNK)], idx_v)
                pltpu.sync_copy(rowst_hbm.at[:, pl.ds(base, CHUNK)], rowst_v)
                for e16 in range(CHUNK):
                    ev = jnp.full((16,), e16, jnp.int32)
                    rows_v[e16, :] = plsc.load_gather(rowst_v, [lane, ev])
                pltpu.sync_copy(rows_v, acc.at[idx_v], add=True)

            return carry

        lax.fori_loop(0, CPW, body, 0)
        plsc.subcore_barrier()
        pltpu.sync_copy(acc.at[pl.ds(s * RPS, RPS)],
                        out_hbm.at[c, pl.ds(s * RPS, RPS)])

    return _gather_rows, _gat_fused, _scatter_add1


BE = 3200


def _mlp_body(eat_ref, xst_ref, w1_ref, b1_ref, w2_ref, b2_ref, w3_ref, b3_ref,
              q_ref, p_ref, out_ref):
    f32 = jnp.float32
    dnum = (((0,), (0,)), ((), ()))
    h1t = jnp.maximum(
        lax.dot_general(w1_ref[...], eat_ref[...], dnum,
                        preferred_element_type=f32) + b1_ref[...], 0.0)
    h2t = jnp.maximum(
        lax.dot_general(w2_ref[...], h1t, dnum,
                        preferred_element_type=f32) + b2_ref[...], 0.0)
    wt = lax.dot_general(w3_ref[...], h2t, dnum,
                         preferred_element_type=f32) + b3_ref[...]
    xrt = lax.dot_general(q_ref[...], xst_ref[...], dnum,
                          preferred_element_type=f32)
    out_ref[...] = lax.dot_general(p_ref[...], wt * xrt, dnum,
                                   preferred_element_type=f32)


_edge_mlp = pl.pallas_call(
    _mlp_body,
    grid=(E // BE,),
    in_specs=[
        pl.BlockSpec((F_EDGE, BE), lambda i: (0, i)),
        pl.BlockSpec((F_IN, BE), lambda i: (0, i)),
        pl.BlockSpec((F_EDGE, 128), lambda i: (0, 0)),
        pl.BlockSpec((128, 1), lambda i: (0, 0)),
        pl.BlockSpec((128, 64), lambda i: (0, 0)),
        pl.BlockSpec((64, 1), lambda i: (0, 0)),
        pl.BlockSpec((64, F_IN * HID), lambda i: (0, 0)),
        pl.BlockSpec((F_IN * HID, 1), lambda i: (0, 0)),
        pl.BlockSpec((F_IN, F_IN * HID), lambda i: (0, 0)),
        pl.BlockSpec((F_IN * HID, HID), lambda i: (0, 0)),
    ],
    out_specs=pl.BlockSpec((F_IN, BE), lambda i: (0, i)),
    out_shape=jax.ShapeDtypeStruct((F_IN, E), jnp.float32),
)


def _node_body(x_ref, a0_ref, a1_ref, wr_ref, bc_ref, wg_ref, asv_ref, adv_ref,
               xw_ref, as_ref, ad_ref, c_ref, num0_ref, den0_ref):
    f32 = jnp.float32
    agg = a0_ref[...] + a1_ref[...]
    x1 = jnp.maximum(
        jnp.dot(x_ref[...], wr_ref[...], preferred_element_type=f32) + agg + bc_ref[...],
        0.0)
    xw = jnp.dot(x1, wg_ref[...], preferred_element_type=f32)
    a_s = jnp.sum(xw * asv_ref[...], axis=1, keepdims=True)
    a_d = jnp.sum(xw * adv_ref[...], axis=1, keepdims=True)
    cval = jnp.max(a_s) + jnp.max(a_d)
    z = a_s + a_d
    ps = jnp.exp(jnp.where(z >= 0, z, 0.2 * z) - cval)
    xw_ref[...] = xw
    as_ref[...] = a_s
    ad_ref[...] = a_d
    c_ref[...] = jnp.reshape(cval, (1, 1))
    num0_ref[...] = ps * xw
    den0_ref[...] = ps


_node_stage = pl.pallas_call(
    _node_body,
    out_shape=(
        jax.ShapeDtypeStruct((N, HID), jnp.float32),
        jax.ShapeDtypeStruct((N, 1), jnp.float32),
        jax.ShapeDtypeStruct((N, 1), jnp.float32),
        jax.ShapeDtypeStruct((1, 1), jnp.float32),
        jax.ShapeDtypeStruct((N, HID), jnp.float32),
        jax.ShapeDtypeStruct((N, 1), jnp.float32),
    ),
)


def _final_body(n0_ref, n1_ref, n2_ref, d0_ref, d1_ref, d2_ref, bg_ref, wf_ref,
                bf_ref, out_ref):
    f32 = jnp.float32
    num = n0_ref[...] + n1_ref[...] + n2_ref[...]
    den = d0_ref[...] + d1_ref[...] + d2_ref[...]
    out_g = num / den + bg_ref[...]
    x2 = jnp.maximum(out_g, 0.0)
    out_ref[...] = jnp.maximum(
        jnp.dot(x2, wf_ref[...], preferred_element_type=f32) + bf_ref[...], 0.0)


_final_stage = pl.pallas_call(
    _final_body,
    out_shape=jax.ShapeDtypeStruct((N, OUT), jnp.float32),
)


def kernel(x, edge_index, edge_attr, batch, W1, b1, W2, b2, W3, b3,
           W_root, b_conv, W_gat, att_src, att_dst, b_gat, W_fc1, b_fc1):
    _gather_rows, _gat_fused, _scatter_add1 = _sc_kernels()
    src = edge_index[0]
    dst = edge_index[1]
    zeros_n = jnp.zeros((N, F_IN), jnp.float32)
    jcol = jnp.arange(F_IN * HID)
    qmat = (jnp.arange(F_IN)[:, None] == (jcol[None, :] // HID)).astype(jnp.float32)
    pmat = ((jcol[:, None] % HID) == jnp.arange(HID)[None, :]).astype(jnp.float32)

    xst = _gather_rows(x, src)
    msgt = _edge_mlp(edge_attr.T, xst, W1, b1.reshape(-1, 1),
                     W2, b2.reshape(-1, 1), W3, b3.reshape(-1, 1), qmat, pmat)
    aggp = _scatter_add1(zeros_n, dst, msgt)
    xw, a_s, a_d, cvl, num0, den0 = _node_stage(
        x, aggp[0], aggp[1], W_root, b_conv.reshape(1, -1), W_gat,
        att_src.reshape(1, -1), att_dst.reshape(1, -1))
    c_rep = jnp.broadcast_to(cvl.reshape(1), (16,))
    nump, denp = _gat_fused(xw, a_s.reshape(N), a_d.reshape(N), src, dst,
                            c_rep, zeros_n)
    out = _final_stage(num0, nump[0], nump[1], den0, denp[0], denp[1],
                       b_gat.reshape(1, -1), W_fc1, b_fc1.reshape(1, -1))
    return out


# speedup vs baseline: 14.3560x; 1.2057x over previous
import functools

import jax
import jax.numpy as jnp
from jax import lax
from jax.experimental import pallas as pl
from jax.experimental.pallas import tpu as pltpu
from jax.experimental.pallas import tpu_sc as plsc

N = 10000
E = 160000
F_IN = 16
F_EDGE = 16
HID = 16
OUT = 64

NC = 2
NS = 16
NW = NC * NS
CHUNK = 128
NCHUNKS = E // CHUNK
CPW = -(-NCHUNKS // NW)
RPS = N // NS

def _worker_id():
    return lax.axis_index("s") * NC + lax.axis_index("c")


@functools.cache
def _sc_kernels():
    mesh = plsc.VectorSubcoreMesh(
        core_axis_name="c", subcore_axis_name="s", num_cores=NC, num_subcores=NS)

    WAY = 4

    @functools.partial(
        pl.kernel,
        out_type=jax.ShapeDtypeStruct((F_IN, E), jnp.float32),
        mesh=mesh,
        compiler_params=pltpu.CompilerParams(use_tc_tiling_on_sc=False, needs_layout_passes=False),
        scratch_types=(
            [pltpu.VMEM((CHUNK,), jnp.int32) for _ in range(WAY)]
            + [pltpu.VMEM((CHUNK, F_IN), jnp.float32) for _ in range(WAY)]
            + [pltpu.VMEM((F_IN, CHUNK), jnp.float32) for _ in range(WAY)]
            + [pltpu.SemaphoreType.DMA((3 * WAY,))]
        ),
    )
    def _gather_rows(x_hbm, idx_hbm, out_hbm, *scr):
        idx_v = scr[0:WAY]
        rows_v = scr[WAY:2 * WAY]
        buf_t = scr[2 * WAY:3 * WAY]
        sems = scr[3 * WAY]
        w = _worker_id()
        lane = lax.iota(jnp.int32, 16)

        def body(j, carry):
            cids = [w + NW * (WAY * j + k) for k in range(WAY)]
            bases = [cid * CHUNK for cid in cids]
            for k in range(WAY):
                @pl.when(cids[k] < NCHUNKS)
                def _(k=k):
                    pltpu.async_copy(idx_hbm.at[pl.ds(bases[k], CHUNK)],
                                     idx_v[k], sems.at[k])
            for k in range(WAY):
                @pl.when(cids[k] < NCHUNKS)
                def _(k=k):
                    pltpu.make_async_copy(idx_hbm.at[pl.ds(bases[k], CHUNK)],
                                          idx_v[k], sems.at[k]).wait()
                    pltpu.async_copy(x_hbm.at[idx_v[k]], rows_v[k],
                                     sems.at[WAY + k])
            for k in range(WAY):
                @pl.when(cids[k] < NCHUNKS)
                def _(k=k):
                    pltpu.make_async_copy(x_hbm.at[idx_v[k]], rows_v[k],
                                          sems.at[WAY + k]).wait()
                    for f in range(F_IN):
                        fv = jnp.full((16,), f, jnp.int32)
                        for g in range(CHUNK // 16):
                            col = plsc.load_gather(rows_v[k], [lane + g * 16, fv])
                            buf_t[k][f, pl.ds(g * 16, 16)] = col
                    pltpu.async_copy(buf_t[k],
                                     out_hbm.at[:, pl.ds(bases[k], CHUNK)],
                                     sems.at[2 * WAY + k])
            for k in range(WAY):
                @pl.when(cids[k] < NCHUNKS)
                def _(k=k):
                    pltpu.make_async_copy(buf_t[k],
                                          out_hbm.at[:, pl.ds(bases[k], CHUNK)],
                                          sems.at[2 * WAY + k]).wait()
            return carry

        lax.fori_loop(0, CPW // WAY, body, 0)

    @functools.partial(
        pl.kernel,
        out_type=(
            jax.ShapeDtypeStruct((NC, N, HID), jnp.float32),
            jax.ShapeDtypeStruct((NC, N, HID), jnp.float32),
        ),
        mesh=mesh,
        compiler_params=pltpu.CompilerParams(use_tc_tiling_on_sc=False, needs_layout_passes=False),
        scratch_types=(
            [pltpu.VMEM((N,), jnp.float32),
             pltpu.VMEM((N,), jnp.float32),
             pltpu.VMEM((16,), jnp.float32)]
            + [pltpu.VMEM((CHUNK,), jnp.int32) for _ in range(8)]
            + [pltpu.VMEM((CHUNK, HID), jnp.float32) for _ in range(8)]
            + [pltpu.VMEM_SHARED((N, HID), jnp.float32),
               pltpu.VMEM_SHARED((N, HID), jnp.float32),
               pltpu.SemaphoreType.DMA((20,))]
        ),
    )
    def _gat_fused(xw_hbm, as_hbm, ad_hbm, src_hbm, dst_hbm, c_hbm, zeros_hbm,
                   numo_hbm, deno_hbm, *scr):
        a_s_v, a_d_v, c_v = scr[0:3]
        idxs_v = scr[3:7]
        idxd_v = scr[7:11]
        rows_v = scr[11:15]
        prep_v = scr[15:19]
        num_acc, den_acc, sems = scr[19:22]
        c = lax.axis_index("c")
        s = lax.axis_index("s")
        w = s * NC + c
        pltpu.sync_copy(zeros_hbm.at[pl.ds(s * RPS, RPS)], num_acc.at[pl.ds(s * RPS, RPS)])
        pltpu.sync_copy(zeros_hbm.at[pl.ds(s * RPS, RPS)], den_acc.at[pl.ds(s * RPS, RPS)])
        pltpu.sync_copy(as_hbm, a_s_v)
        pltpu.sync_copy(ad_hbm, a_d_v)
        pltpu.sync_copy(c_hbm, c_v)
        plsc.subcore_barrier()
        cvec = c_v[...]

        def body(j, carry):
            cids = [w + NW * (4 * j + k) for k in range(4)]
            bases = [cid * CHUNK for cid in cids]
            for k in range(4):
                @pl.when(cids[k] < NCHUNKS)
                def _(k=k):
                    pltpu.async_copy(src_hbm.at[pl.ds(bases[k], CHUNK)],
                                     idxs_v[k], sems.at[k])
                    pltpu.async_copy(dst_hbm.at[pl.ds(bases[k], CHUNK)],
                                     idxd_v[k], sems.at[4 + k])
            for k in range(4):
                @pl.when(cids[k] < NCHUNKS)
                def _(k=k):
                    pltpu.make_async_copy(src_hbm.at[pl.ds(bases[k], CHUNK)],
                                          idxs_v[k], sems.at[k]).wait()
                    pltpu.async_copy(xw_hbm.at[idxs_v[k]], rows_v[k],
                                     sems.at[8 + k])
            for k in range(4):
                @pl.when(cids[k] < NCHUNKS)
                def _(k=k):
                    pltpu.make_async_copy(dst_hbm.at[pl.ds(bases[k], CHUNK)],
                                          idxd_v[k], sems.at[4 + k]).wait()
                    pltpu.make_async_copy(xw_hbm.at[idxs_v[k]], rows_v[k],
                                          sems.at[8 + k]).wait()
                    for g in range(CHUNK // 16):
                        si = idxs_v[k][pl.ds(g * 16, 16)]
                        di = idxd_v[k][pl.ds(g * 16, 16)]
                        z = plsc.load_gather(a_s_v, [si]) + plsc.load_gather(a_d_v, [di])
                        p_vec = jnp.exp(jnp.maximum(z, 0.2 * z) - cvec)
                        for e16 in range(16):
                            pv = p_vec[e16]
                            row = g * 16 + e16
                            rows_v[k][row, :] = rows_v[k][row, :] * pv
                            prep_v[k][row, :] = jnp.full((16,), pv, jnp.float32)
                    pltpu.async_copy(rows_v[k], num_acc.at[idxd_v[k]],
                                     sems.at[12 + k], add=True)
                    pltpu.async_copy(prep_v[k], den_acc.at[idxd_v[k]],
                                     sems.at[16 + k], add=True)
            for k in range(4):
                @pl.when(cids[k] < NCHUNKS)
                def _(k=k):
                    pltpu.make_async_copy(rows_v[k], num_acc.at[idxd_v[k]],
                                          sems.at[12 + k]).wait()
                    pltpu.make_async_copy(prep_v[k], den_acc.at[idxd_v[k]],
                                          sems.at[16 + k]).wait()
            return carry

        lax.fori_loop(0, CPW // 4, body, 0)
        plsc.subcore_barrier()
        pltpu.sync_copy(num_acc.at[pl.ds(s * RPS, RPS)],
                        numo_hbm.at[c, pl.ds(s * RPS, RPS)])
        pltpu.sync_copy(den_acc.at[pl.ds(s * RPS, RPS)],
                        deno_hbm.at[c, pl.ds(s * RPS, RPS)])

    @functools.partial(
        pl.kernel,
        out_type=jax.ShapeDtypeStruct((NC, N, F_IN), jnp.float32),
        mesh=mesh,
        compiler_params=pltpu.CompilerParams(use_tc_tiling_on_sc=False, needs_layout_passes=False),
        scratch_types=(
            [pltpu.VMEM((CHUNK,), jnp.int32) for _ in range(4)]
            + [pltpu.VMEM((F_IN, CHUNK), jnp.float32) for _ in range(4)]
            + [pltpu.VMEM((CHUNK, F_IN), jnp.float32) for _ in range(4)]
            + [pltpu.VMEM_SHARED((N, F_IN), jnp.float32),
               pltpu.SemaphoreType.DMA((12,))]
        ),
    )
    def _scatter_add1(zeros_hbm, dst_hbm, rowst_hbm, out_hbm, *scr):
        idx_v = scr[0:4]
        rowst_v = scr[4:8]
        rows_v = scr[8:12]
        acc = scr[12]
        sems = scr[13]
        c = lax.axis_index("c")
        s = lax.axis_index("s")
        w = s * NC + c
        lane = lax.iota(jnp.int32, 16)
        pltpu.sync_copy(zeros_hbm.at[pl.ds(s * RPS, RPS)], acc.at[pl.ds(s * RPS, RPS)])
        plsc.subcore_barrier()

        def body(j, carry):
            cids = [w + NW * (4 * j + k) for k in range(4)]
            bases = [cid * CHUNK for cid in cids]
            for k in range(4):
                @pl.when(cids[k] < NCHUNKS)
                def _(k=k):
                    pltpu.async_copy(dst_hbm.at[pl.ds(bases[k], CHUNK)],
                                     idx_v[k], sems.at[k])
                    pltpu.async_copy(rowst_hbm.at[:, pl.ds(bases[k], CHUNK)],
                                     rowst_v[k], sems.at[4 + k])
            for k in range(4):
                @pl.when(cids[k] < NCHUNKS)
                def _(k=k):
                    pltpu.make_async_copy(dst_hbm.at[pl.ds(bases[k], CHUNK)],
                                          idx_v[k], sems.at[k]).wait()
                    pltpu.make_async_copy(rowst_hbm.at[:, pl.ds(bases[k], CHUNK)],
                                          rowst_v[k], sems.at[4 + k]).wait()
                    for e16 in range(CHUNK):
                        ev = jnp.full((16,), e16, jnp.int32)
                        rows_v[k][e16, :] = plsc.load_gather(rowst_v[k], [lane, ev])
                    pltpu.async_copy(rows_v[k], acc.at[idx_v[k]], sems.at[8 + k],
                                     add=True)
            for k in range(4):
                @pl.when(cids[k] < NCHUNKS)
                def _(k=k):
                    pltpu.make_async_copy(rows_v[k], acc.at[idx_v[k]],
                                          sems.at[8 + k]).wait()
            return carry

        lax.fori_loop(0, CPW // 4, body, 0)
        plsc.subcore_barrier()
        pltpu.sync_copy(acc.at[pl.ds(s * RPS, RPS)],
                        out_hbm.at[c, pl.ds(s * RPS, RPS)])

    return _gather_rows, _gat_fused, _scatter_add1


BE = 3200


def _mlp_body(eat_ref, xst_ref, w1_ref, b1_ref, w2_ref, b2_ref, w3_ref, b3_ref,
              q_ref, p_ref, out_ref):
    f32 = jnp.float32
    dnum = (((0,), (0,)), ((), ()))
    h1t = jnp.maximum(
        lax.dot_general(w1_ref[...], eat_ref[...], dnum,
                        preferred_element_type=f32) + b1_ref[...], 0.0)
    h2t = jnp.maximum(
        lax.dot_general(w2_ref[...], h1t, dnum,
                        preferred_element_type=f32) + b2_ref[...], 0.0)
    wt = lax.dot_general(w3_ref[...], h2t, dnum,
                         preferred_element_type=f32) + b3_ref[...]
    xrt = lax.dot_general(q_ref[...], xst_ref[...], dnum,
                          preferred_element_type=f32)
    out_ref[...] = lax.dot_general(p_ref[...], wt * xrt, dnum,
                                   preferred_element_type=f32)


_edge_mlp = pl.pallas_call(
    _mlp_body,
    grid=(E // BE,),
    in_specs=[
        pl.BlockSpec((F_EDGE, BE), lambda i: (0, i)),
        pl.BlockSpec((F_IN, BE), lambda i: (0, i)),
        pl.BlockSpec((F_EDGE, 128), lambda i: (0, 0)),
        pl.BlockSpec((128, 1), lambda i: (0, 0)),
        pl.BlockSpec((128, 64), lambda i: (0, 0)),
        pl.BlockSpec((64, 1), lambda i: (0, 0)),
        pl.BlockSpec((64, F_IN * HID), lambda i: (0, 0)),
        pl.BlockSpec((F_IN * HID, 1), lambda i: (0, 0)),
        pl.BlockSpec((F_IN, F_IN * HID), lambda i: (0, 0)),
        pl.BlockSpec((F_IN * HID, HID), lambda i: (0, 0)),
    ],
    out_specs=pl.BlockSpec((F_IN, BE), lambda i: (0, i)),
    out_shape=jax.ShapeDtypeStruct((F_IN, E), jnp.float32),
)


def _node_body(x_ref, a0_ref, a1_ref, wr_ref, bc_ref, wg_ref, asv_ref, adv_ref,
               xw_ref, as_ref, ad_ref, c_ref, num0_ref, den0_ref):
    f32 = jnp.float32
    agg = a0_ref[...] + a1_ref[...]
    x1 = jnp.maximum(
        jnp.dot(x_ref[...], wr_ref[...], preferred_element_type=f32) + agg + bc_ref[...],
        0.0)
    xw = jnp.dot(x1, wg_ref[...], preferred_element_type=f32)
    a_s = jnp.sum(xw * asv_ref[...], axis=1, keepdims=True)
    a_d = jnp.sum(xw * adv_ref[...], axis=1, keepdims=True)
    cval = jnp.max(a_s) + jnp.max(a_d)
    z = a_s + a_d
    ps = jnp.exp(jnp.where(z >= 0, z, 0.2 * z) - cval)
    xw_ref[...] = xw
    as_ref[...] = a_s
    ad_ref[...] = a_d
    c_ref[...] = jnp.reshape(cval, (1, 1))
    num0_ref[...] = ps * xw
    den0_ref[...] = ps


_node_stage = pl.pallas_call(
    _node_body,
    out_shape=(
        jax.ShapeDtypeStruct((N, HID), jnp.float32),
        jax.ShapeDtypeStruct((N, 1), jnp.float32),
        jax.ShapeDtypeStruct((N, 1), jnp.float32),
        jax.ShapeDtypeStruct((1, 1), jnp.float32),
        jax.ShapeDtypeStruct((N, HID), jnp.float32),
        jax.ShapeDtypeStruct((N, 1), jnp.float32),
    ),
)


def _final_body(n0_ref, n1_ref, n2_ref, d0_ref, d1_ref, d2_ref, bg_ref, wf_ref,
                bf_ref, out_ref):
    f32 = jnp.float32
    num = n0_ref[...] + n1_ref[...] + n2_ref[...]
    den = d0_ref[...] + d1_ref[...] + d2_ref[...]
    out_g = num / den + bg_ref[...]
    x2 = jnp.maximum(out_g, 0.0)
    out_ref[...] = jnp.maximum(
        jnp.dot(x2, wf_ref[...], preferred_element_type=f32) + bf_ref[...], 0.0)


_final_stage = pl.pallas_call(
    _final_body,
    out_shape=jax.ShapeDtypeStruct((N, OUT), jnp.float32),
)


def kernel(x, edge_index, edge_attr, batch, W1, b1, W2, b2, W3, b3,
           W_root, b_conv, W_gat, att_src, att_dst, b_gat, W_fc1, b_fc1):
    _gather_rows, _gat_fused, _scatter_add1 = _sc_kernels()
    src = edge_index[0]
    dst = edge_index[1]
    zeros_n = jnp.zeros((N, F_IN), jnp.float32)
    jcol = jnp.arange(F_IN * HID)
    qmat = (jnp.arange(F_IN)[:, None] == (jcol[None, :] // HID)).astype(jnp.float32)
    pmat = ((jcol[:, None] % HID) == jnp.arange(HID)[None, :]).astype(jnp.float32)

    xst = _gather_rows(x, src)
    msgt = _edge_mlp(edge_attr.T, xst, W1, b1.reshape(-1, 1),
                     W2, b2.reshape(-1, 1), W3, b3.reshape(-1, 1), qmat, pmat)
    aggp = _scatter_add1(zeros_n, dst, msgt)
    xw, a_s, a_d, cvl, num0, den0 = _node_stage(
        x, aggp[0], aggp[1], W_root, b_conv.reshape(1, -1), W_gat,
        att_src.reshape(1, -1), att_dst.reshape(1, -1))
    c_rep = jnp.broadcast_to(cvl.reshape(1), (16,))
    nump, denp = _gat_fused(xw, a_s.reshape(N), a_d.reshape(N), src, dst,
                            c_rep, zeros_n)
    out = _final_stage(num0, nump[0], nump[1], den0, denp[0], denp[1],
                       b_gat.reshape(1, -1), W_fc1, b_fc1.reshape(1, -1))
    return out
